# Initial kernel scaffold; baseline (speedup 1.0000x reference)
import jax
import jax.numpy as jnp
from jax.experimental import pallas as pl


def kernel(feat, W_conv, b_conv, W_obj, b_obj, W_dlt, b_dlt):
    raise NotImplementedError("write your pallas kernel here")


# speedup vs baseline: 10.3756x; 10.3756x over previous
import numpy as np
import jax
import jax.numpy as jnp
from jax import lax
from jax.experimental import pallas as pl
from jax.experimental.pallas import tpu as pltpu

_N, _C, _H, _W = 2, 256, 64, 64
_A = 3
_STRIDE = 16
_SIZES = (64.0, 128.0, 256.0)
_IMG = 1024.0
_NMS_T = 0.7
_K = 1000
_SCALE_CLAMP = float(np.log(1000.0 / 16.0))
_HW = _H * _W
_NA = _HW * _A
_R = _NA // 128
_M = 1024

_HIGH = lax.Precision.HIGHEST


_CH = 512


def _conv_head_kernel(xl_ref, xm_ref, xr_ref, wk_ref, bc_ref, wh_ref, bh_ref,
                      out_ref):
    xs = (xl_ref, xm_ref, xr_ref)

    def body(c, _):
        base = c * _CH
        acc = jnp.zeros((_CH, _C), jnp.float32)
        for ky in range(3):
            for kx in range(3):
                x = xs[kx][0, pl.ds(base + 64 * ky, _CH), :]
                acc = acc + jnp.dot(x, wk_ref[ky * 3 + kx],
                                    preferred_element_type=jnp.float32)
        t = jnp.maximum(acc + bc_ref[...], 0.0)
        out_ref[0, pl.ds(base, _CH), :] = jnp.dot(
            t.astype(jnp.bfloat16), wh_ref[...],
            preferred_element_type=jnp.float32) + bh_ref[...]
        return 0

    lax.fori_loop(0, _HW // _CH, body, 0)


def _decode_boxes(a_x0, a_y0, a_x1, a_y1, d_x, d_y, d_w, d_h):
    aw = a_x1 - a_x0
    ah = a_y1 - a_y0
    acx = a_x0 + 0.5 * aw
    acy = a_y0 + 0.5 * ah
    d_w = jnp.minimum(d_w, _SCALE_CLAMP)
    d_h = jnp.minimum(d_h, _SCALE_CLAMP)
    pcx = d_x * aw + acx
    pcy = d_y * ah + acy
    pw = jnp.exp(d_w) * aw
    ph = jnp.exp(d_h) * ah
    x0 = jnp.clip(pcx - 0.5 * pw, 0.0, _IMG)
    y0 = jnp.clip(pcy - 0.5 * ph, 0.0, _IMG)
    x1 = jnp.clip(pcx + 0.5 * pw, 0.0, _IMG)
    y1 = jnp.clip(pcy + 0.5 * ph, 0.0, _IMG)
    return x0, y0, x1, y1


def _select_nms_kernel(sc_ref, data_ref, fb_ref, fs_ref,
                       sel_ref, pos_ref, acc_ref, box_ref, s_ref, keep_ref):
    f32 = jnp.float32
    scores = sc_ref[0]
    bits = lax.bitcast_convert_type(scores, jnp.int32)
    key = jnp.where(bits >= 0, bits, bits ^ jnp.int32(0x7FFFFFFF))

    hi = key >= 0
    cnt_hi = jnp.sum(hi.astype(jnp.int32))
    sel_hi = cnt_hi >= _K
    k_rem = jnp.where(sel_hi, jnp.int32(_K), jnp.int32(_K) - cnt_hi)
    in_sel = hi == sel_hi
    low31 = key & jnp.int32(0x7FFFFFFF)

    def radix_body(t, p):
        cand = p | (jnp.int32(1) << (jnp.int32(30) - t))
        c = jnp.sum((in_sel & (low31 >= cand)).astype(jnp.int32))
        return jnp.where(c >= k_rem, cand, p)

    p = lax.fori_loop(0, 31, radix_body, jnp.int32(0))
    kth = jnp.where(sel_hi, p, p + jnp.int32(-2147483648))

    gt = key > kth
    eq = key == kth
    m0 = jnp.sum(gt.astype(jnp.int32))
    needf = (jnp.int32(_K) - m0).astype(f32)

    li = lax.broadcasted_iota(jnp.int32, (128, 128), 0)
    lj = lax.broadcasted_iota(jnp.int32, (128, 128), 1)
    tl128 = (li <= lj).astype(f32)
    ri = lax.broadcasted_iota(jnp.int32, (_R, _R), 0)
    rj = lax.broadcasted_iota(jnp.int32, (_R, _R), 1)
    slr = (rj < ri).astype(f32)

    def csum_flat(x):
        within = jnp.dot(x, tl128, preferred_element_type=f32, precision=_HIGH)
        rows = jnp.sum(x, axis=1, keepdims=True)
        pref = jnp.dot(slr, rows, preferred_element_type=f32, precision=_HIGH)
        return within + pref

    eqr = csum_flat(eq.astype(f32))
    sel = gt | (eq & (eqr <= needf))
    self_f = sel.astype(f32)
    pos = csum_flat(self_f) - 1.0
    sel_ref[...] = self_f
    pos_ref[...] = pos

    acc_ref[...] = jnp.zeros((_M, 16), f32)
    d_iota = lax.broadcasted_iota(jnp.int32, (_M, 1), 0).astype(f32)

    def comp_body(r, _):
        pr = pos_ref[pl.ds(r, 1), :]
        sr = sel_ref[pl.ds(r, 1), :]
        oh = jnp.where((d_iota == pr) & (sr > 0.5), 1.0, 0.0)
        blk = data_ref[0, pl.ds(r * 128, 128), :]
        acc_ref[...] += jnp.dot(oh, blk, preferred_element_type=f32,
                                precision=_HIGH)
        return 0

    lax.fori_loop(0, _R, comp_body, 0)
    comp = acc_ref[...]
    comp_t = jnp.transpose(comp)
    lane_m = lax.broadcasted_iota(jnp.int32, (1, _M), 1)

    s_col = comp[:, 0:1]
    s_row = comp_t[0:1, :]
    mi = lax.broadcasted_iota(jnp.int32, (_M, _M), 0)
    mj = lax.broadcasted_iota(jnp.int32, (_M, _M), 1)
    before = ((s_col > s_row) | ((s_col == s_row) & (mi < mj))) & (mi < _K)
    rank_row = jnp.sum(before.astype(f32), axis=0, keepdims=True)
    rank_row = jnp.where(lane_m < _K, rank_row, jnp.float32(_M - 1))
    perm = (d_iota == rank_row).astype(f32)
    srt = jnp.dot(perm, comp, preferred_element_type=f32, precision=_HIGH)
    srt_t = jnp.transpose(srt)

    x0, y0, x1, y1 = _decode_boxes(
        srt[:, 1:2], srt[:, 2:3], srt[:, 3:4], srt[:, 4:5],
        srt[:, 5:6], srt[:, 6:7], srt[:, 7:8], srt[:, 8:9])
    box_ref[...] = jnp.concatenate([x0, y0, x1, y1], axis=1)
    x0r, y0r, x1r, y1r = _decode_boxes(
        srt_t[1:2, :], srt_t[2:3, :], srt_t[3:4, :], srt_t[4:5, :],
        srt_t[5:6, :], srt_t[6:7, :], srt_t[7:8, :], srt_t[8:9, :])
    area_r = (x1r - x0r) * (y1r - y0r)

    def iou_body(rb, _):
        lo = rb * 128
        bb = box_ref[pl.ds(lo, 128), :]
        x0b, y0b = bb[:, 0:1], bb[:, 1:2]
        x1b, y1b = bb[:, 2:3], bb[:, 3:4]
        iw = jnp.maximum(jnp.minimum(x1b, x1r) - jnp.maximum(x0b, x0r), 0.0)
        ih = jnp.maximum(jnp.minimum(y1b, y1r) - jnp.maximum(y0b, y0r), 0.0)
        inter = iw * ih
        area_b = (x1b - x0b) * (y1b - y0b)
        iou = inter / (area_b + area_r - inter + 1e-9)
        ib = lax.broadcasted_iota(jnp.int32, (128, _M), 0) + lo
        jb = lax.broadcasted_iota(jnp.int32, (128, _M), 1)
        s_ref[pl.ds(lo, 128), :] = ((iou > _NMS_T) & (jb > ib)).astype(f32)
        return 0

    lax.fori_loop(0, _M // 128, iou_body, 0)

    keep_ref[...] = jnp.ones((1, _M), f32)

    def nms_body(i, _):
        kv = keep_ref[...]
        ki = jnp.sum(jnp.where(lane_m == i, kv, 0.0))
        row = s_ref[pl.ds(i, 1), :]
        keep_ref[...] = kv * (1.0 - ki * row)
        return 0

    lax.fori_loop(0, _K, nms_body, 0)

    keep_row = keep_ref[...]
    validf = (lane_m < _K).astype(f32)
    kf = keep_row * validf
    nk = (1.0 - keep_row) * validf
    tlm = (mi <= mj).astype(f32)
    cs_k = jnp.dot(kf, tlm, preferred_element_type=f32, precision=_HIGH)
    cs_n = jnp.dot(nk, tlm, preferred_element_type=f32, precision=_HIGH)
    mtot = jnp.sum(kf)
    dest = jnp.where(kf > 0.5, cs_k - 1.0, mtot + cs_n - 1.0)
    dest = jnp.where(validf > 0.5, dest, jnp.float32(_M - 1))
    fperm = (d_iota == dest).astype(f32)
    fb_full = jnp.dot(fperm, box_ref[...], preferred_element_type=f32,
                      precision=_HIGH)
    fs_raw = jnp.dot(fperm, srt[:, 0:1], preferred_element_type=f32,
                     precision=_HIGH)
    fs_full = jnp.where(d_iota < mtot, fs_raw, jnp.float32(-1e9))
    fb_ref[0] = fb_full[0:_K, :]
    fs_ref[0] = fs_full[0:_K, :]


def _make_anchors():
    ys = jnp.arange(_H, dtype=jnp.float32) * _STRIDE
    xs = jnp.arange(_W, dtype=jnp.float32) * _STRIDE
    sy, sx = jnp.meshgrid(ys, xs, indexing="ij")
    shifts = jnp.stack([sx.ravel(), sy.ravel(), sx.ravel(), sy.ravel()], axis=1)
    base = jnp.array([[-s / 2.0, -s / 2.0, s / 2.0, s / 2.0] for s in _SIZES],
                     dtype=jnp.float32)
    return (shifts[:, None, :] + base[None, :, :]).reshape(-1, 4)


def kernel(feat, W_conv, b_conv, W_obj, b_obj, W_dlt, b_dlt):
    f = feat.transpose(0, 2, 3, 1)
    fl = jnp.pad(f, ((0, 0), (0, 0), (1, 0), (0, 0)))[:, :, :_W, :]
    fr = jnp.pad(f, ((0, 0), (0, 0), (0, 1), (0, 0)))[:, :, 1:, :]
    vpad = ((0, 0), (64, 64), (0, 0))
    bf16 = jnp.bfloat16
    xl = jnp.pad(fl.reshape(_N, _HW, _C), vpad).astype(bf16)
    xm = jnp.pad(f.reshape(_N, _HW, _C), vpad).astype(bf16)
    xr = jnp.pad(fr.reshape(_N, _HW, _C), vpad).astype(bf16)

    wk = W_conv.transpose(2, 3, 1, 0).reshape(9, _C, _C).astype(bf16)
    bc2 = b_conv.reshape(1, _C)
    wh = jnp.concatenate(
        [W_obj.reshape(_A, _C).T, W_dlt.reshape(4 * _A, _C).T,
         jnp.zeros((_C, 1), jnp.float32)], axis=1).astype(bf16)
    bh2 = jnp.concatenate([b_obj, b_dlt, jnp.zeros((1,), jnp.float32)]
                          ).reshape(1, 16)

    head = pl.pallas_call(
        _conv_head_kernel,
        grid=(_N,),
        in_specs=[
            pl.BlockSpec((1, _HW + 128, _C), lambda n: (n, 0, 0)),
            pl.BlockSpec((1, _HW + 128, _C), lambda n: (n, 0, 0)),
            pl.BlockSpec((1, _HW + 128, _C), lambda n: (n, 0, 0)),
            pl.BlockSpec((9, _C, _C), lambda n: (0, 0, 0)),
            pl.BlockSpec((1, _C), lambda n: (0, 0)),
            pl.BlockSpec((_C, 16), lambda n: (0, 0)),
            pl.BlockSpec((1, 16), lambda n: (0, 0)),
        ],
        out_specs=pl.BlockSpec((1, _HW, 16), lambda n: (n, 0, 0)),
        out_shape=jax.ShapeDtypeStruct((_N, _HW, 16), jnp.float32),
    )(xl, xm, xr, wk, bc2, wh, bh2)

    scores_flat = head[:, :, 0:_A].reshape(_N, _NA)
    scores2d = scores_flat.reshape(_N, _R, 128)
    deltas = head[:, :, _A:_A + 4 * _A].reshape(_N, _NA, 4)
    anchors = jnp.broadcast_to(_make_anchors()[None], (_N, _NA, 4))
    data = jnp.concatenate(
        [scores_flat[..., None], anchors, deltas,
         jnp.zeros((_N, _NA, 7), jnp.float32)], axis=-1)

    fb, fs = pl.pallas_call(
        _select_nms_kernel,
        grid=(_N,),
        in_specs=[
            pl.BlockSpec((1, _R, 128), lambda n: (n, 0, 0)),
            pl.BlockSpec((1, _NA, 16), lambda n: (n, 0, 0)),
        ],
        out_specs=[
            pl.BlockSpec((1, _K, 4), lambda n: (n, 0, 0)),
            pl.BlockSpec((1, _K, 1), lambda n: (n, 0, 0)),
        ],
        out_shape=[
            jax.ShapeDtypeStruct((_N, _K, 4), jnp.float32),
            jax.ShapeDtypeStruct((_N, _K, 1), jnp.float32),
        ],
        scratch_shapes=[
            pltpu.VMEM((_R, 128), jnp.float32),
            pltpu.VMEM((_R, 128), jnp.float32),
            pltpu.VMEM((_M, 16), jnp.float32),
            pltpu.VMEM((_M, 4), jnp.float32),
            pltpu.VMEM((_M, _M), jnp.float32),
            pltpu.VMEM((1, _M), jnp.float32),
        ],
    )(scores2d, data)
    return fb, fs.reshape(_N, _K)
